# Initial kernel scaffold; baseline (speedup 1.0000x reference)
import jax
import jax.numpy as jnp
from jax.experimental import pallas as pl


def kernel(x):
    raise NotImplementedError("write your pallas kernel here")


# speedup vs baseline: 9.5782x; 9.5782x over previous
import functools

import jax
import jax.numpy as jnp
from jax import lax
from jax.experimental import pallas as pl
from jax.experimental.pallas import tpu as pltpu

K = 16
BR = 256


def _knn_body(xr_ref, xt_ref, src_ref, dst_ref, *, M, K):
    n = pl.program_id(0)
    rb = pl.program_id(1)

    xr = xr_ref[0]
    xt = xt_ref[0]
    sq_all = jnp.sum(xt * xt, axis=0, keepdims=True)

    mm = jax.lax.dot_general(
        xr, xt, (((1,), (0,)), ((), ())),
        preferred_element_type=jnp.float32)
    sq_r = jnp.sum(xr * xr, axis=1, keepdims=True)
    d = sq_r + sq_all - 2.0 * mm

    col = lax.broadcasted_iota(jnp.int32, d.shape, 1)
    big = jnp.float32(jnp.inf)

    idxs = []
    for _ in range(K):
        m = jnp.min(d, axis=1, keepdims=True)
        eq = d == m
        idx = jnp.min(jnp.where(eq, col, M), axis=1, keepdims=True)
        idxs.append(idx)
        d = jnp.where(col == idx, big, d)

    nbr = jnp.concatenate(idxs, axis=1)
    offset = n * M
    src_ref[0] = nbr + offset
    row0 = rb * BR + offset
    dst_ref[0] = (lax.broadcasted_iota(jnp.int32, (BR, K), 0) + row0)


@functools.partial(jax.jit, static_argnames=())
def kernel(x):
    if x.ndim == 2:
        x = x[None, :, :]
    N, M, D = x.shape
    xt = x.transpose(0, 2, 1)

    grid = (N, M // BR)
    src, dst = pl.pallas_call(
        functools.partial(_knn_body, M=M, K=K),
        grid=grid,
        in_specs=[
            pl.BlockSpec((1, BR, D), lambda n, r: (n, r, 0)),
            pl.BlockSpec((1, D, M), lambda n, r: (n, 0, 0)),
        ],
        out_specs=[
            pl.BlockSpec((1, BR, K), lambda n, r: (n, r, 0)),
            pl.BlockSpec((1, BR, K), lambda n, r: (n, r, 0)),
        ],
        out_shape=[
            jax.ShapeDtypeStruct((N, M, K), jnp.int32),
            jax.ShapeDtypeStruct((N, M, K), jnp.int32),
        ],
    )(x, xt)
    return src.reshape(-1), dst.reshape(-1)


# speedup vs baseline: 12.8497x; 1.3416x over previous
import functools

import jax
import jax.numpy as jnp
from jax import lax
from jax.experimental import pallas as pl
from jax.experimental.pallas import tpu as pltpu

K = 16
BR = 256
NS = 16
NL = 128


def _oddeven_network(n):
    pairs = []

    def merge(lo, m, r):
        step = r * 2
        if step < m:
            merge(lo, m, step)
            merge(lo + r, m, step)
            for i in range(lo + r, lo + m - r, step):
                pairs.append((i, i + r))
        else:
            pairs.append((lo, lo + r))

    def sort(lo, m):
        if m > 1:
            h = m // 2
            sort(lo, h)
            sort(lo + h, h)
            merge(lo, m, 1)

    sort(0, n)
    return pairs


_NETWORK = _oddeven_network(NS)


def _knn_body(xr_ref, xt_ref, src_ref, dst_ref, *, M, K):
    n = pl.program_id(0)
    rb = pl.program_id(1)

    xr = xr_ref[0]
    xt = xt_ref[0]
    sq_all = jnp.sum(xt * xt, axis=0, keepdims=True)

    mm = jax.lax.dot_general(
        xr, xt, (((1,), (0,)), ((), ())),
        preferred_element_type=jnp.float32)
    sq_r = jnp.sum(xr * xr, axis=1, keepdims=True)
    d = jnp.maximum(sq_r + sq_all - 2.0 * mm, 0.0)
    bits = jax.lax.bitcast_convert_type(d, jnp.int32)

    A = [(bits[:, s * NL:(s + 1) * NL] & jnp.int32(~0xF)) | jnp.int32(s)
         for s in range(NS)]

    for i, j in _NETWORK:
        lo = jnp.minimum(A[i], A[j])
        hi = jnp.maximum(A[i], A[j])
        A[i], A[j] = lo, hi

    lane = lax.broadcasted_iota(jnp.int32, (BR, NL), 1)
    fill = jnp.int32(0x7FFFFFFF)
    cols = []
    for _ in range(K):
        heads = A[0]
        m = jnp.min(heads, axis=1, keepdims=True)
        eq = heads == m
        l_idx = jnp.min(jnp.where(eq, lane, NL), axis=1, keepdims=True)
        win = eq & (lane == l_idx)
        for s in range(NS - 1):
            A[s] = jnp.where(win, A[s + 1], A[s])
        A[NS - 1] = jnp.where(win, fill, A[NS - 1])
        cols.append((m & 0xF) * NL + l_idx)

    nbr = jnp.concatenate(cols, axis=1)
    offset = n * M
    src_ref[0] = nbr + offset
    row0 = rb * BR + offset
    dst_ref[0] = lax.broadcasted_iota(jnp.int32, (BR, K), 0) + row0


@jax.jit
def kernel(x):
    if x.ndim == 2:
        x = x[None, :, :]
    N, M, D = x.shape
    xt = x.transpose(0, 2, 1)

    grid = (N, M // BR)
    src, dst = pl.pallas_call(
        functools.partial(_knn_body, M=M, K=K),
        grid=grid,
        in_specs=[
            pl.BlockSpec((1, BR, D), lambda n, r: (n, r, 0)),
            pl.BlockSpec((1, D, M), lambda n, r: (n, 0, 0)),
        ],
        out_specs=[
            pl.BlockSpec((1, BR, K), lambda n, r: (n, r, 0)),
            pl.BlockSpec((1, BR, K), lambda n, r: (n, r, 0)),
        ],
        out_shape=[
            jax.ShapeDtypeStruct((N, M, K), jnp.int32),
            jax.ShapeDtypeStruct((N, M, K), jnp.int32),
        ],
    )(x, xt)
    return src.reshape(-1), dst.reshape(-1)


# speedup vs baseline: 20.5006x; 1.5954x over previous
import functools

import jax
import jax.numpy as jnp
from jax import lax
from jax.experimental import pallas as pl
from jax.experimental.pallas import tpu as pltpu

K = 16
BR = 256
NS = 16
NL = 128
SPLITS = 2


def _oddeven_network(n):
    pairs = []

    def merge(lo, m, r):
        step = r * 2
        if step < m:
            merge(lo, m, step)
            merge(lo + r, m, step)
            for i in range(lo + r, lo + m - r, step):
                pairs.append((i, i + r))
        else:
            pairs.append((lo, lo + r))

    def sort(lo, m):
        if m > 1:
            h = m // 2
            sort(lo, h)
            sort(lo + h, h)
            merge(lo, m, 1)

    sort(0, n)
    return pairs


_NETWORK = _oddeven_network(NS)


def _knn_body(xr_ref, xt_ref, src_ref, dst_ref, *, M, K):
    n = pl.program_id(0)
    rb = pl.program_id(1)

    xr = xr_ref[0]
    xt = xt_ref[0]
    sq_all = jnp.sum(xt * xt, axis=0, keepdims=True)

    mm = jax.lax.dot_general(
        xr, xt, (((1,), (0,)), ((), ())),
        preferred_element_type=jnp.float32)
    sq_r = jnp.sum(xr * xr, axis=1, keepdims=True)
    d = jnp.maximum(sq_r + sq_all - 2.0 * mm, 0.0)
    bits = jax.lax.bitcast_convert_type(d, jnp.int32)

    A = [jax.lax.bitcast_convert_type(
            ((bits[:, s * NL:(s + 1) * NL] & jnp.int32(~0xF)) | jnp.int32(s))
            + jnp.int32(1 << 23),
            jnp.float32)
         for s in range(NS)]

    for i, j in _NETWORK:
        lo = jnp.minimum(A[i], A[j])
        hi = jnp.maximum(A[i], A[j])
        A[i], A[j] = lo, hi

    BRS = BR // SPLITS
    lanef = lax.broadcasted_iota(jnp.int32, (BRS, NL), 1).astype(jnp.float32)
    fill = jnp.float32(jnp.inf)
    parts = [[a[p * BRS:(p + 1) * BRS] for a in A] for p in range(SPLITS)]
    ms = [[] for _ in range(SPLITS)]
    ls = [[] for _ in range(SPLITS)]
    for _ in range(K):
        for p in range(SPLITS):
            Ap = parts[p]
            heads = Ap[0]
            m = jnp.min(heads, axis=1, keepdims=True)
            eq = heads == m
            lf = jnp.min(jnp.where(eq, lanef, jnp.float32(NL)),
                         axis=1, keepdims=True)
            win = eq & (lanef == lf)
            for s in range(NS - 1):
                Ap[s] = jnp.where(win, Ap[s + 1], Ap[s])
            Ap[NS - 1] = jnp.where(win, fill, Ap[NS - 1])
            ms[p].append(m)
            ls[p].append(lf)

    rows = []
    for p in range(SPLITS):
        mk = jax.lax.bitcast_convert_type(
            jnp.concatenate(ms[p], axis=1), jnp.int32)
        lk = jnp.concatenate(ls[p], axis=1).astype(jnp.int32)
        rows.append((mk & 0xF) * NL + lk)
    nbr = jnp.concatenate(rows, axis=0)
    offset = n * M
    src_ref[0] = nbr + offset
    row0 = rb * BR + offset
    dst_ref[0] = lax.broadcasted_iota(jnp.int32, (BR, K), 0) + row0


@jax.jit
def kernel(x):
    if x.ndim == 2:
        x = x[None, :, :]
    N, M, D = x.shape
    xt = x.transpose(0, 2, 1)

    grid = (N, M // BR)
    src, dst = pl.pallas_call(
        functools.partial(_knn_body, M=M, K=K),
        grid=grid,
        in_specs=[
            pl.BlockSpec((1, BR, D), lambda n, r: (n, r, 0)),
            pl.BlockSpec((1, D, M), lambda n, r: (n, 0, 0)),
        ],
        out_specs=[
            pl.BlockSpec((1, BR, K), lambda n, r: (n, r, 0)),
            pl.BlockSpec((1, BR, K), lambda n, r: (n, r, 0)),
        ],
        out_shape=[
            jax.ShapeDtypeStruct((N, M, K), jnp.int32),
            jax.ShapeDtypeStruct((N, M, K), jnp.int32),
        ],
    )(x, xt)
    return src.reshape(-1), dst.reshape(-1)


# speedup vs baseline: 20.5029x; 1.0001x over previous
import functools

import jax
import jax.numpy as jnp
from jax import lax
from jax.experimental import pallas as pl
from jax.experimental.pallas import tpu as pltpu

K = 16
BR = 256
NS = 16
NL = 128
SPLITS = 4


def _oddeven_network(n):
    pairs = []

    def merge(lo, m, r):
        step = r * 2
        if step < m:
            merge(lo, m, step)
            merge(lo + r, m, step)
            for i in range(lo + r, lo + m - r, step):
                pairs.append((i, i + r))
        else:
            pairs.append((lo, lo + r))

    def sort(lo, m):
        if m > 1:
            h = m // 2
            sort(lo, h)
            sort(lo + h, h)
            merge(lo, m, 1)

    sort(0, n)
    return pairs


_NETWORK = _oddeven_network(NS)


def _knn_body(xr_ref, xt_ref, src_ref, dst_ref, *, M, K):
    n = pl.program_id(0)
    rb = pl.program_id(1)

    xr = xr_ref[0]
    xt = xt_ref[0]
    sq_all = jnp.sum(xt * xt, axis=0, keepdims=True)

    mm = jax.lax.dot_general(
        xr, xt, (((1,), (0,)), ((), ())),
        preferred_element_type=jnp.float32)
    sq_r = jnp.sum(xr * xr, axis=1, keepdims=True)
    d = jnp.maximum(sq_r + sq_all - 2.0 * mm, 0.0)
    bits = jax.lax.bitcast_convert_type(d, jnp.int32)

    A = [jax.lax.bitcast_convert_type(
            ((bits[:, s * NL:(s + 1) * NL] & jnp.int32(~0xF)) | jnp.int32(s))
            + jnp.int32(1 << 23),
            jnp.float32)
         for s in range(NS)]

    for i, j in _NETWORK:
        lo = jnp.minimum(A[i], A[j])
        hi = jnp.maximum(A[i], A[j])
        A[i], A[j] = lo, hi

    BRS = BR // SPLITS
    lanef = lax.broadcasted_iota(jnp.int32, (BRS, NL), 1).astype(jnp.float32)
    fill = jnp.float32(jnp.inf)
    parts = [[a[p * BRS:(p + 1) * BRS] for a in A] for p in range(SPLITS)]
    ms = [[] for _ in range(SPLITS)]
    ls = [[] for _ in range(SPLITS)]
    for _ in range(K):
        for p in range(SPLITS):
            Ap = parts[p]
            heads = Ap[0]
            m = jnp.min(heads, axis=1, keepdims=True)
            eq = heads == m
            lf = jnp.min(jnp.where(eq, lanef, jnp.float32(NL)),
                         axis=1, keepdims=True)
            win = eq & (lanef == lf)
            for s in range(NS - 1):
                Ap[s] = jnp.where(win, Ap[s + 1], Ap[s])
            Ap[NS - 1] = jnp.where(win, fill, Ap[NS - 1])
            ms[p].append(m)
            ls[p].append(lf)

    rows = []
    for p in range(SPLITS):
        mk = jax.lax.bitcast_convert_type(
            jnp.concatenate(ms[p], axis=1), jnp.int32)
        lk = jnp.concatenate(ls[p], axis=1).astype(jnp.int32)
        rows.append((mk & 0xF) * NL + lk)
    nbr = jnp.concatenate(rows, axis=0)
    offset = n * M
    src_ref[0] = nbr + offset
    row0 = rb * BR + offset
    dst_ref[0] = lax.broadcasted_iota(jnp.int32, (BR, K), 0) + row0


@jax.jit
def kernel(x):
    if x.ndim == 2:
        x = x[None, :, :]
    N, M, D = x.shape
    xt = x.transpose(0, 2, 1)

    grid = (N, M // BR)
    src, dst = pl.pallas_call(
        functools.partial(_knn_body, M=M, K=K),
        grid=grid,
        in_specs=[
            pl.BlockSpec((1, BR, D), lambda n, r: (n, r, 0)),
            pl.BlockSpec((1, D, M), lambda n, r: (n, 0, 0)),
        ],
        out_specs=[
            pl.BlockSpec((1, BR, K), lambda n, r: (n, r, 0)),
            pl.BlockSpec((1, BR, K), lambda n, r: (n, r, 0)),
        ],
        out_shape=[
            jax.ShapeDtypeStruct((N, M, K), jnp.int32),
            jax.ShapeDtypeStruct((N, M, K), jnp.int32),
        ],
    )(x, xt)
    return src.reshape(-1), dst.reshape(-1)


# speedup vs baseline: 21.0157x; 1.0250x over previous
import functools

import jax
import jax.numpy as jnp
from jax import lax
from jax.experimental import pallas as pl
from jax.experimental.pallas import tpu as pltpu

K = 16
BR = 256
NS = 16
NL = 128
SPLITS = 4


def _oddeven_network(n):
    pairs = []

    def merge(lo, m, r):
        step = r * 2
        if step < m:
            merge(lo, m, step)
            merge(lo + r, m, step)
            for i in range(lo + r, lo + m - r, step):
                pairs.append((i, i + r))
        else:
            pairs.append((lo, lo + r))

    def sort(lo, m):
        if m > 1:
            h = m // 2
            sort(lo, h)
            sort(lo + h, h)
            merge(lo, m, 1)

    sort(0, n)
    return pairs


_NETWORK = _oddeven_network(NS)


def _knn_body(xr_ref, xt_ref, src_ref, dst_ref, *, M, K):
    n = pl.program_id(0)
    rb = pl.program_id(1)

    xr = xr_ref[0]
    xt = xt_ref[0]
    sq_all = jnp.sum(xt * xt, axis=0, keepdims=True)

    mm = jax.lax.dot_general(
        xr, xt, (((1,), (0,)), ((), ())),
        preferred_element_type=jnp.float32)
    sq_r = jnp.sum(xr * xr, axis=1, keepdims=True)
    d = jnp.maximum(sq_r + sq_all - 2.0 * mm, 1e-30)
    bits = jax.lax.bitcast_convert_type(d, jnp.int32)

    A = [jax.lax.bitcast_convert_type(
            (bits[:, s * NL:(s + 1) * NL] & jnp.int32(~0xF)) | jnp.int32(s),
            jnp.float32)
         for s in range(NS)]

    for i, j in _NETWORK:
        lo = jnp.minimum(A[i], A[j])
        hi = jnp.maximum(A[i], A[j])
        A[i], A[j] = lo, hi

    BRS = BR // SPLITS
    lanef = lax.broadcasted_iota(jnp.int32, (BRS, NL), 1).astype(jnp.float32)
    parts = [[a[p * BRS:(p + 1) * BRS] for a in A] for p in range(SPLITS)]
    ms = [[] for _ in range(SPLITS)]
    ls = [[] for _ in range(SPLITS)]
    for k in range(K):
        for p in range(SPLITS):
            Ap = parts[p]
            heads = Ap[0]
            m = jnp.min(heads, axis=1, keepdims=True)
            eq = heads == m
            lf = jnp.min(jnp.where(eq, lanef, jnp.float32(NL)),
                         axis=1, keepdims=True)
            win = eq & (lanef == lf)
            for s in range(NS - 1 - k):
                Ap[s] = jnp.where(win, Ap[s + 1], Ap[s])
            ms[p].append(m)
            ls[p].append(lf)

    rows = []
    for p in range(SPLITS):
        mk = jax.lax.bitcast_convert_type(
            jnp.concatenate(ms[p], axis=1), jnp.int32)
        lk = jnp.concatenate(ls[p], axis=1).astype(jnp.int32)
        rows.append((mk & 0xF) * NL + lk)
    nbr = jnp.concatenate(rows, axis=0)
    offset = n * M
    src_ref[0] = nbr + offset
    row0 = rb * BR + offset
    dst_ref[0] = lax.broadcasted_iota(jnp.int32, (BR, K), 0) + row0


@jax.jit
def kernel(x):
    if x.ndim == 2:
        x = x[None, :, :]
    N, M, D = x.shape
    xt = x.transpose(0, 2, 1)

    grid = (N, M // BR)
    src, dst = pl.pallas_call(
        functools.partial(_knn_body, M=M, K=K),
        grid=grid,
        in_specs=[
            pl.BlockSpec((1, BR, D), lambda n, r: (n, r, 0)),
            pl.BlockSpec((1, D, M), lambda n, r: (n, 0, 0)),
        ],
        out_specs=[
            pl.BlockSpec((1, BR, K), lambda n, r: (n, r, 0)),
            pl.BlockSpec((1, BR, K), lambda n, r: (n, r, 0)),
        ],
        out_shape=[
            jax.ShapeDtypeStruct((N, M, K), jnp.int32),
            jax.ShapeDtypeStruct((N, M, K), jnp.int32),
        ],
    )(x, xt)
    return src.reshape(-1), dst.reshape(-1)
